# baseline (speedup 1.0000x reference)
import jax
import jax.numpy as jnp
from jax.experimental import pallas as pl
from jax.experimental.pallas import tpu as pltpu

N, F_IN, D = 10000, 256, 128
CIN = F_IN + D
CH = 1000
NC = N // CH
S = 3


def _cell_body(x_hbm, h_hbm, wz_ref, bz_ref, wr_ref, br_ref, wh_ref, bh_ref,
               lw_ref, lb_ref, out_hbm, H_hbm,
               xbuf, hbuf, Hbuf, obuf, wbig,
               sx, sh, sH, so):
    def in_copies(i, s):
        rows = pl.ds(i * CH, CH)
        return (pltpu.make_async_copy(x_hbm.at[rows], xbuf.at[s], sx.at[s]),
                pltpu.make_async_copy(h_hbm.at[rows], hbuf.at[s], sh.at[s]))

    def out_copies(i, s):
        rows = pl.ds(i * CH, CH)
        return (pltpu.make_async_copy(Hbuf.at[s], H_hbm.at[rows], sH.at[s]),
                pltpu.make_async_copy(obuf.at[s], out_hbm.at[rows], so.at[s]))

    for j in range(S - 1):
        for c in in_copies(j, j % S):
            c.start()

    wbig[:, 0:D] = (wz_ref[0, 0] + wz_ref[1, 0]).astype(jnp.bfloat16)
    wbig[:, D:2 * D] = (wr_ref[0, 0] + wr_ref[1, 0]).astype(jnp.bfloat16)
    wbig[:, 2 * D:3 * D] = (wh_ref[0, 0] + wh_ref[1, 0]).astype(jnp.bfloat16)

    def dot(a, b):
        return jax.lax.dot_general(a, b, (((1,), (0,)), ((), ())),
                                   preferred_element_type=jnp.float32)

    for i in range(NC):
        s = i % S
        if i + S - 1 < NC:
            for c in in_copies(i + S - 1, (i + S - 1) % S):
                c.start()
        for c in in_copies(i, s):
            c.wait()
        if i >= S:
            for c in out_copies(i - S, s):
                c.wait()

        hb = hbuf[s]
        xb = xbuf[s].astype(jnp.bfloat16)
        hb16 = hb.astype(jnp.bfloat16)
        acc_x = dot(xb, wbig[:F_IN, :])
        acc_h = dot(hb16, wbig[F_IN:, :2 * D])
        bz = bz_ref[...].reshape(1, D)
        br = br_ref[...].reshape(1, D)
        bh = bh_ref[...].reshape(1, D)
        z = jax.nn.sigmoid(acc_x[:, 0:D] + acc_h[:, 0:D] + bz)
        r = jax.nn.sigmoid(acc_x[:, D:2 * D] + acc_h[:, D:2 * D] + br)
        ht = jnp.tanh(acc_x[:, 2 * D:3 * D]
                      + dot((r * hb).astype(jnp.bfloat16), wbig[F_IN:, 2 * D:3 * D])
                      + bh)
        Hb = z * hb + (1.0 - z) * ht
        Hbuf[s] = Hb
        relu = jnp.maximum(Hb, 0.0)
        obuf[s] = jnp.sum(relu * lw_ref[...], axis=1, keepdims=True) + lb_ref[...].reshape(1, 1)

        for c in out_copies(i, s):
            c.start()

    for i in range(max(0, NC - S), NC):
        for c in out_copies(i, i % S):
            c.wait()


@jax.jit
def _run(x, h, W_z, b_z, W_r, b_r, W_h, b_h, lin_w, lin_b):
    hbm = pl.BlockSpec(memory_space=pltpu.MemorySpace.HBM)
    vmem = pl.BlockSpec(memory_space=pltpu.MemorySpace.VMEM)
    out, H = pl.pallas_call(
        _cell_body,
        in_specs=[hbm, hbm, vmem, vmem, vmem, vmem, vmem, vmem, vmem, vmem],
        out_specs=[hbm, hbm],
        out_shape=[
            jax.ShapeDtypeStruct((N, 1), jnp.float32),
            jax.ShapeDtypeStruct((N, D), jnp.float32),
        ],
        scratch_shapes=[
            pltpu.MemorySpace.VMEM((S, CH, F_IN), jnp.float32),
            pltpu.MemorySpace.VMEM((S, CH, D), jnp.float32),
            pltpu.MemorySpace.VMEM((S, CH, D), jnp.float32),
            pltpu.MemorySpace.VMEM((S, CH, 1), jnp.float32),
            pltpu.MemorySpace.VMEM((CIN, 3 * D), jnp.bfloat16),
            pltpu.SemaphoreType.DMA((S,)),
            pltpu.SemaphoreType.DMA((S,)),
            pltpu.SemaphoreType.DMA((S,)),
            pltpu.SemaphoreType.DMA((S,)),
        ],
    )(x, h, W_z, b_z, W_r, b_r, W_h, b_h, lin_w, lin_b)
    return out, H


def kernel(x, edge_index, edge_attr, h, W_z, b_z, W_r, b_r, W_h, b_h,
           lin_w, lin_b):
    del edge_index, edge_attr
    return _run(x, h, W_z, b_z, W_r, b_r, W_h, b_h, lin_w, lin_b)


# speedup vs baseline: 1.0704x; 1.0704x over previous
import jax
import jax.numpy as jnp
from jax.experimental import pallas as pl
from jax.experimental.pallas import tpu as pltpu

N, F_IN, D = 10000, 256, 128
CIN = F_IN + D
CH = 2000
NC = N // CH
S = 3


def _cell_body(x_hbm, h_hbm, wz_ref, bz_ref, wr_ref, br_ref, wh_ref, bh_ref,
               lw_ref, lb_ref, out_hbm, H_hbm,
               xbuf, hbuf, Hbuf, obuf, wbig,
               sx, sh, sH, so):
    def in_copies(i, s):
        rows = pl.ds(i * CH, CH)
        return (pltpu.make_async_copy(x_hbm.at[rows], xbuf.at[s], sx.at[s]),
                pltpu.make_async_copy(h_hbm.at[rows], hbuf.at[s], sh.at[s]))

    def out_copies(i, s):
        rows = pl.ds(i * CH, CH)
        return (pltpu.make_async_copy(Hbuf.at[s], H_hbm.at[rows], sH.at[s]),
                pltpu.make_async_copy(obuf.at[s], out_hbm.at[rows], so.at[s]))

    for j in range(S - 1):
        for c in in_copies(j, j % S):
            c.start()

    wbig[:, 0:D] = (wz_ref[0, 0] + wz_ref[1, 0]).astype(jnp.bfloat16)
    wbig[:, D:2 * D] = (wr_ref[0, 0] + wr_ref[1, 0]).astype(jnp.bfloat16)
    wbig[:, 2 * D:3 * D] = (wh_ref[0, 0] + wh_ref[1, 0]).astype(jnp.bfloat16)

    def dot(a, b):
        return jax.lax.dot_general(a, b, (((1,), (0,)), ((), ())),
                                   preferred_element_type=jnp.float32)

    for i in range(NC):
        s = i % S
        if i + S - 1 < NC:
            for c in in_copies(i + S - 1, (i + S - 1) % S):
                c.start()
        for c in in_copies(i, s):
            c.wait()
        if i >= S:
            for c in out_copies(i - S, s):
                c.wait()

        hb = hbuf[s]
        xb = xbuf[s].astype(jnp.bfloat16)
        hb16 = hb.astype(jnp.bfloat16)
        acc_x = dot(xb, wbig[:F_IN, :])
        acc_h = dot(hb16, wbig[F_IN:, :2 * D])
        bz = bz_ref[...].reshape(1, D)
        br = br_ref[...].reshape(1, D)
        bh = bh_ref[...].reshape(1, D)
        z = jax.nn.sigmoid(acc_x[:, 0:D] + acc_h[:, 0:D] + bz)
        r = jax.nn.sigmoid(acc_x[:, D:2 * D] + acc_h[:, D:2 * D] + br)
        ht = jnp.tanh(acc_x[:, 2 * D:3 * D]
                      + dot((r * hb).astype(jnp.bfloat16), wbig[F_IN:, 2 * D:3 * D])
                      + bh)
        Hb = z * hb + (1.0 - z) * ht
        Hbuf[s] = Hb
        relu = jnp.maximum(Hb, 0.0)
        obuf[s] = jnp.sum(relu * lw_ref[...], axis=1, keepdims=True) + lb_ref[...].reshape(1, 1)

        for c in out_copies(i, s):
            c.start()

    for i in range(max(0, NC - S), NC):
        for c in out_copies(i, i % S):
            c.wait()


@jax.jit
def _run(x, h, W_z, b_z, W_r, b_r, W_h, b_h, lin_w, lin_b):
    hbm = pl.BlockSpec(memory_space=pltpu.MemorySpace.HBM)
    vmem = pl.BlockSpec(memory_space=pltpu.MemorySpace.VMEM)
    out, H = pl.pallas_call(
        _cell_body,
        in_specs=[hbm, hbm, vmem, vmem, vmem, vmem, vmem, vmem, vmem, vmem],
        out_specs=[hbm, hbm],
        out_shape=[
            jax.ShapeDtypeStruct((N, 1), jnp.float32),
            jax.ShapeDtypeStruct((N, D), jnp.float32),
        ],
        scratch_shapes=[
            pltpu.MemorySpace.VMEM((S, CH, F_IN), jnp.float32),
            pltpu.MemorySpace.VMEM((S, CH, D), jnp.float32),
            pltpu.MemorySpace.VMEM((S, CH, D), jnp.float32),
            pltpu.MemorySpace.VMEM((S, CH, 1), jnp.float32),
            pltpu.MemorySpace.VMEM((CIN, 3 * D), jnp.bfloat16),
            pltpu.SemaphoreType.DMA((S,)),
            pltpu.SemaphoreType.DMA((S,)),
            pltpu.SemaphoreType.DMA((S,)),
            pltpu.SemaphoreType.DMA((S,)),
        ],
    )(x, h, W_z, b_z, W_r, b_r, W_h, b_h, lin_w, lin_b)
    return out, H


def kernel(x, edge_index, edge_attr, h, W_z, b_z, W_r, b_r, W_h, b_h,
           lin_w, lin_b):
    del edge_index, edge_attr
    return _run(x, h, W_z, b_z, W_r, b_r, W_h, b_h, lin_w, lin_b)
